# baseline (speedup 1.0000x reference)
import jax
import jax.numpy as jnp
from jax.experimental import pallas as pl
from jax.experimental.pallas import tpu as pltpu


def _fwd_kernel(ids_ref, emb_ref, w1_ref, b1_ref, w2_ref, b2_ref,
                w3_ref, b3_ref, out_ref, t_ref):
    f32 = jnp.float32
    nf, tb = ids_ref.shape
    v, d = emb_ref.shape

    @pl.when(pl.program_id(0) == 0)
    def _precompute_tables():
        emb = emb_ref[...]
        for i in range(nf):
            w1_i = w1_ref[:, i * d:(i + 1) * d]
            t_ref[i * v:(i + 1) * v, :] = jax.lax.dot_general(
                emb, w1_i, (((1,), (1,)), ((), ())),
                preferred_element_type=f32)

    ids = ids_ref[...]
    sub_iota = jax.lax.broadcasted_iota(jnp.int32, (v, tb), 0)
    oh = jnp.concatenate(
        [(ids[i:i + 1, :] == sub_iota).astype(f32) for i in range(nf)],
        axis=0)
    acc = jax.lax.dot_general(oh, t_ref[...], (((0,), (0,)), ((), ())),
                              preferred_element_type=f32)

    x = acc + b1_ref[...]
    alpha = 1.6732632423543772
    scale = 1.0507009873554805
    h1 = scale * jnp.where(x > 0, x, alpha * (jnp.exp(x) - 1.0))
    h2 = jnp.tanh(
        jax.lax.dot_general(h1, w2_ref[...], (((1,), (1,)), ((), ())),
                            preferred_element_type=f32) + b2_ref[...])
    o8 = jax.lax.dot_general(h2, w3_ref[...], (((1,), (1,)), ((), ())),
                             preferred_element_type=f32) + b3_ref[0, 0]
    out_ref[...] = o8


def kernel(vocab_ids, pos_emb, W1, b1, W2, b2, W3, b3):
    nf, b = vocab_ids.shape
    v, d = pos_emb.shape
    h = W1.shape[0]
    ids = vocab_ids.astype(jnp.int32)
    w3p = jnp.zeros((8, W3.shape[1]), jnp.float32).at[0:1].set(W3)
    tb = 2048 if b % 2048 == 0 else b
    nb = b // tb
    o8 = pl.pallas_call(
        _fwd_kernel,
        grid=(nb,),
        in_specs=[
            pl.BlockSpec((nf, tb), lambda i: (0, i)),
            pl.BlockSpec(pos_emb.shape, lambda i: (0, 0)),
            pl.BlockSpec(W1.shape, lambda i: (0, 0)),
            pl.BlockSpec((1, b1.shape[0]), lambda i: (0, 0)),
            pl.BlockSpec(W2.shape, lambda i: (0, 0)),
            pl.BlockSpec((1, b2.shape[0]), lambda i: (0, 0)),
            pl.BlockSpec(w3p.shape, lambda i: (0, 0)),
            pl.BlockSpec((1, 1), lambda i: (0, 0)),
        ],
        out_specs=pl.BlockSpec((tb, 8), lambda i: (i, 0)),
        out_shape=jax.ShapeDtypeStruct((b, 8), jnp.float32),
        scratch_shapes=[pltpu.VMEM((nf * v, h), jnp.float32)],
    )(ids, pos_emb, W1, b1.reshape(1, -1), W2, b2.reshape(1, -1),
      w3p, b3.reshape(1, 1))
    return o8[:, 0:1]


# speedup vs baseline: 1.0003x; 1.0003x over previous
import jax
import jax.numpy as jnp
from jax.experimental import pallas as pl
from jax.experimental.pallas import tpu as pltpu


def _fwd_kernel(ids_ref, emb_ref, w1_ref, b1_ref, w2_ref, b2_ref,
                w3_ref, b3_ref, out_ref, t_ref):
    f32 = jnp.float32
    nf, tb = ids_ref.shape
    v, d = emb_ref.shape

    @pl.when(pl.program_id(0) == 0)
    def _precompute_tables():
        emb = emb_ref[...]
        for i in range(nf):
            w1_i = w1_ref[:, i * d:(i + 1) * d]
            t_ref[i * v:(i + 1) * v, :] = jax.lax.dot_general(
                emb, w1_i, (((1,), (1,)), ((), ())),
                preferred_element_type=f32)

    ids = ids_ref[...]
    sub_iota = jax.lax.broadcasted_iota(jnp.int32, (v, tb), 0)
    oh = jnp.concatenate(
        [(ids[i:i + 1, :] == sub_iota).astype(f32) for i in range(nf)],
        axis=0)
    acc = jax.lax.dot_general(oh, t_ref[...], (((0,), (0,)), ((), ())),
                              preferred_element_type=f32)

    x = acc + b1_ref[...]
    alpha = 1.6732632423543772
    scale = 1.0507009873554805
    h1 = scale * jnp.where(x > 0, x, alpha * (jnp.exp(x) - 1.0))
    h2 = jnp.tanh(
        jax.lax.dot_general(h1, w2_ref[...], (((1,), (1,)), ((), ())),
                            preferred_element_type=f32) + b2_ref[...])
    o8 = jax.lax.dot_general(h2, w3_ref[...], (((1,), (1,)), ((), ())),
                             preferred_element_type=f32) + b3_ref[0, 0]
    out_ref[...] = o8[:, 0:1]


def kernel(vocab_ids, pos_emb, W1, b1, W2, b2, W3, b3):
    nf, b = vocab_ids.shape
    v, d = pos_emb.shape
    h = W1.shape[0]
    ids = vocab_ids.astype(jnp.int32)
    w3p = jnp.zeros((8, W3.shape[1]), jnp.float32).at[0:1].set(W3)
    tb = 2048 if b % 2048 == 0 else b
    nb = b // tb
    o8 = pl.pallas_call(
        _fwd_kernel,
        grid=(nb,),
        in_specs=[
            pl.BlockSpec((nf, tb), lambda i: (0, i)),
            pl.BlockSpec(pos_emb.shape, lambda i: (0, 0)),
            pl.BlockSpec(W1.shape, lambda i: (0, 0)),
            pl.BlockSpec((1, b1.shape[0]), lambda i: (0, 0)),
            pl.BlockSpec(W2.shape, lambda i: (0, 0)),
            pl.BlockSpec((1, b2.shape[0]), lambda i: (0, 0)),
            pl.BlockSpec(w3p.shape, lambda i: (0, 0)),
            pl.BlockSpec((1, 1), lambda i: (0, 0)),
        ],
        out_specs=pl.BlockSpec((tb, 1), lambda i: (i, 0)),
        out_shape=jax.ShapeDtypeStruct((b, 1), jnp.float32),
        scratch_shapes=[pltpu.VMEM((nf * v, h), jnp.float32)],
    )(ids, pos_emb, W1, b1.reshape(1, -1), W2, b2.reshape(1, -1),
      w3p, b3.reshape(1, 1))
    return o8


# speedup vs baseline: 1.2618x; 1.2614x over previous
import jax
import jax.numpy as jnp
from jax.experimental import pallas as pl
from jax.experimental.pallas import tpu as pltpu


def _fwd_kernel(ids_ref, emb_ref, w1_ref, b1_ref, w2_ref, b2_ref,
                w3_ref, b3_ref, out_ref, t_ref):
    f32 = jnp.float32
    nf, tb = ids_ref.shape
    v, d = emb_ref.shape

    @pl.when(pl.program_id(0) == 0)
    def _precompute_tables():
        emb = emb_ref[...]
        for i in range(nf):
            w1_i = w1_ref[:, i * d:(i + 1) * d]
            t_ref[i * v:(i + 1) * v, :] = jax.lax.dot_general(
                emb, w1_i, (((1,), (1,)), ((), ())),
                preferred_element_type=f32).astype(jnp.bfloat16)

    ids = ids_ref[...]
    sub_iota = jax.lax.broadcasted_iota(jnp.int32, (v, tb), 0)
    oh = jnp.concatenate(
        [(ids[i:i + 1, :] == sub_iota).astype(jnp.bfloat16)
         for i in range(nf)], axis=0)
    acc = jax.lax.dot_general(oh, t_ref[...], (((0,), (0,)), ((), ())),
                              preferred_element_type=f32)

    x = acc + b1_ref[...]
    alpha = 1.6732632423543772
    scale = 1.0507009873554805
    h1 = scale * jnp.where(x > 0, x, alpha * (jnp.exp(x) - 1.0))
    h2 = jnp.tanh(
        jax.lax.dot_general(h1, w2_ref[...], (((1,), (1,)), ((), ())),
                            preferred_element_type=f32) + b2_ref[...])
    out_ref[...] = (jnp.sum(h2 * w3_ref[...], axis=1, keepdims=True)
                    + b3_ref[0, 0])


def kernel(vocab_ids, pos_emb, W1, b1, W2, b2, W3, b3):
    nf, b = vocab_ids.shape
    v, d = pos_emb.shape
    h = W1.shape[0]
    ids = vocab_ids.astype(jnp.int32)
    tb = 2048 if b % 2048 == 0 else b
    nb = b // tb
    o8 = pl.pallas_call(
        _fwd_kernel,
        grid=(nb,),
        in_specs=[
            pl.BlockSpec((nf, tb), lambda i: (0, i)),
            pl.BlockSpec(pos_emb.shape, lambda i: (0, 0)),
            pl.BlockSpec(W1.shape, lambda i: (0, 0)),
            pl.BlockSpec((1, b1.shape[0]), lambda i: (0, 0)),
            pl.BlockSpec(W2.shape, lambda i: (0, 0)),
            pl.BlockSpec((1, b2.shape[0]), lambda i: (0, 0)),
            pl.BlockSpec(W3.shape, lambda i: (0, 0)),
            pl.BlockSpec((1, 1), lambda i: (0, 0)),
        ],
        out_specs=pl.BlockSpec((tb, 1), lambda i: (i, 0)),
        out_shape=jax.ShapeDtypeStruct((b, 1), jnp.float32),
        scratch_shapes=[pltpu.VMEM((nf * v, h), jnp.bfloat16)],
    )(ids, pos_emb, W1, b1.reshape(1, -1), W2, b2.reshape(1, -1),
      W3, b3.reshape(1, 1))
    return o8


# speedup vs baseline: 1.3114x; 1.0394x over previous
import jax
import jax.numpy as jnp
from jax.experimental import pallas as pl
from jax.experimental.pallas import tpu as pltpu


def _fwd_kernel(ids_ref, emb_ref, w1_ref, b1_ref, w2_ref, b2_ref,
                w3_ref, b3_ref, out_ref, t_ref):
    f32 = jnp.float32
    nf, tb = ids_ref.shape
    v, d = emb_ref.shape

    @pl.when(pl.program_id(0) == 0)
    def _precompute_tables():
        emb = emb_ref[...]
        for i in range(nf):
            w1_i = w1_ref[:, i * d:(i + 1) * d]
            t_ref[i * v:(i + 1) * v, :] = jax.lax.dot_general(
                emb, w1_i, (((1,), (1,)), ((), ())),
                preferred_element_type=f32).astype(jnp.bfloat16)

    ids = ids_ref[...]
    sub_iota = jax.lax.broadcasted_iota(jnp.int32, (v, tb), 0)
    oh = jnp.concatenate(
        [(ids[i:i + 1, :] == sub_iota).astype(jnp.bfloat16)
         for i in range(nf)], axis=0)
    acc = jax.lax.dot_general(oh, t_ref[...], (((0,), (0,)), ((), ())),
                              preferred_element_type=f32)

    x = acc + b1_ref[...]
    alpha = 1.6732632423543772
    scale = 1.0507009873554805
    h1 = scale * jnp.where(x > 0, x, alpha * (jnp.exp(x) - 1.0))
    h2 = jnp.tanh(
        jax.lax.dot_general(h1, w2_ref[...], (((1,), (1,)), ((), ())),
                            preferred_element_type=f32) + b2_ref[...])
    out_ref[...] = (jnp.sum(h2 * w3_ref[...], axis=1, keepdims=True)
                    + b3_ref[0, 0])


def kernel(vocab_ids, pos_emb, W1, b1, W2, b2, W3, b3):
    nf, b = vocab_ids.shape
    v, d = pos_emb.shape
    h = W1.shape[0]
    ids = vocab_ids.astype(jnp.int32)
    tb = 4096 if b % 4096 == 0 else b
    nb = b // tb
    o8 = pl.pallas_call(
        _fwd_kernel,
        grid=(nb,),
        in_specs=[
            pl.BlockSpec((nf, tb), lambda i: (0, i)),
            pl.BlockSpec(pos_emb.shape, lambda i: (0, 0)),
            pl.BlockSpec(W1.shape, lambda i: (0, 0)),
            pl.BlockSpec((1, b1.shape[0]), lambda i: (0, 0)),
            pl.BlockSpec(W2.shape, lambda i: (0, 0)),
            pl.BlockSpec((1, b2.shape[0]), lambda i: (0, 0)),
            pl.BlockSpec(W3.shape, lambda i: (0, 0)),
            pl.BlockSpec((1, 1), lambda i: (0, 0)),
        ],
        out_specs=pl.BlockSpec((tb, 1), lambda i: (i, 0)),
        out_shape=jax.ShapeDtypeStruct((b, 1), jnp.float32),
        scratch_shapes=[pltpu.VMEM((nf * v, h), jnp.bfloat16)],
    )(ids, pos_emb, W1, b1.reshape(1, -1), W2, b2.reshape(1, -1),
      W3, b3.reshape(1, 1))
    return o8
